# baseline (speedup 1.0000x reference)
import jax
import jax.numpy as jnp
from jax import lax
from jax.experimental import pallas as pl
from jax.experimental.pallas import tpu as pltpu
from jax.experimental.pallas import tpu_sc as plsc

N = 10000
E = 320000
K = 20
ALPHA = [0.7, 0.7, 0.7 / 2.0, 0.7 / 3.0, 0.7 / 4.0]

NPAD = 10240
NC, NS = 2, 16
CH = 128
HF = 64
NPH = 2
CPH = 79
EPT = NPH * CPH * CH
EPAD = NS * EPT
RPS = NPAD // NS
DUMMY = NPAD - 1
SLOTS = 8
LAG = 4

_mesh = plsc.VectorSubcoreMesh(core_axis_name="c", subcore_axis_name="s")


def _sc_hop_body(z3, rowt, colt, ypart, row_v, col_v,
                 b0, b1, b2, b3, b4, b5, b6, b7,
                 g0, g1, g2, g3, g4, g5, g6, g7,
                 s0, s1, s2, s3, s4, s5, s6, s7,
                 y_sh):
    c = lax.axis_index("c")
    s = lax.axis_index("s")
    bufs = [b0, b1, b2, b3, b4, b5, b6, b7]
    gsem = [g0, g1, g2, g3, g4, g5, g6, g7]
    ssem = [s0, s1, s2, s3, s4, s5, s6, s7]
    ztab = z3.at[c]

    def drain_gather(t):
        pltpu.make_async_copy(ztab.at[pl.ds(0, CH)], bufs[t], gsem[t]).wait()

    def drain_scatter(t):
        pltpu.make_async_copy(bufs[t], y_sh.at[pl.ds(0, CH)], ssem[t]).wait()

    zero16 = jnp.zeros((16,), jnp.float32)

    @pl.loop(0, CH)
    def _zero(i):
        for g in range(HF // 16):
            b0[i, pl.ds(g * 16, 16)] = zero16

    zdesc = [pltpu.async_copy(b0, y_sh.at[pl.ds(s * RPS + q * CH, CH)],
                              ssem[q])
             for q in range(RPS // CH)]
    for dsc in zdesc:
        dsc.wait()
    plsc.subcore_barrier()

    for ph in range(NPH):
        pltpu.sync_copy(rowt.at[s].at[ph], row_v)
        pltpu.sync_copy(colt.at[s].at[ph], col_v)

        @pl.loop(0, (CPH + LAG + SLOTS - 1) // SLOTS + 1)
        def _grp(g):
            for t in range(SLOTS):
                j = g * SLOTS + t

                @pl.when(j <= CPH - 1)
                def _gather():
                    @pl.when(j >= SLOTS)
                    def _w():
                        drain_scatter(t)

                    pltpu.async_copy(ztab.at[row_v.at[j]], bufs[t], gsem[t])

                i = j - LAG
                u = (t - LAG) % SLOTS

                @pl.when(jnp.logical_and(i >= 0, i <= CPH - 1))
                def _scatter():
                    drain_gather(u)
                    pltpu.async_copy(bufs[u], y_sh.at[col_v.at[i]],
                                     ssem[u], add=True)

        for t in range(SLOTS):
            drain_scatter(t)

    plsc.subcore_barrier()
    for q in range(RPS // CH):
        off = s * RPS + q * CH
        pltpu.sync_copy(y_sh.at[pl.ds(off, CH)], b1)
        pltpu.sync_copy(b1, ypart.at[c].at[pl.ds(off, CH)])


_sc_hop = pl.kernel(
    _sc_hop_body,
    out_type=jax.ShapeDtypeStruct((NC, NPAD, HF), jnp.float32),
    mesh=_mesh,
    compiler_params=pltpu.CompilerParams(use_tc_tiling_on_sc=False),
    scratch_types=(
        [
            pltpu.VMEM((CPH, CH), jnp.int32),
            pltpu.VMEM((CPH, CH), jnp.int32),
        ]
        + [pltpu.VMEM((CH, HF), jnp.float32)] * SLOTS
        + [pltpu.SemaphoreType.DMA] * (2 * SLOTS)
        + [pltpu.VMEM_SHARED((NPAD, HF), jnp.float32)]
    ),
)


_RB = 1280
_GRID = NPAD // _RB


def _prep_body(dp, x4st, dvb, dv2b, dvbf, h0, z0):
    b = pl.program_id(0)
    deg = dp[...][:, 0:1] + 1.0
    dv = lax.rsqrt(deg)
    ridx = lax.broadcasted_iota(jnp.int32, (_RB, 1), 0) + b * _RB
    dv = jnp.where(ridx < N, dv, 0.0)
    dvb[...] = jnp.broadcast_to(dv, (_RB, HF))
    dv2b[...] = jnp.broadcast_to(dv * dv, (_RB, HF))
    dvbf[...] = jnp.broadcast_to(dv, (_RB, 128))
    xb = x4st[...][0]
    h0[0] = ALPHA[0] * xb
    z0[0] = dv * xb


_prep = pl.pallas_call(
    _prep_body,
    grid=(_GRID, NC),
    in_specs=[
        pl.BlockSpec((_RB, HF), lambda b, hb: (b, 0)),
        pl.BlockSpec((1, _RB, HF), lambda b, hb: (hb, b, 0)),
    ],
    out_specs=[
        pl.BlockSpec((_RB, HF), lambda b, hb: (b, 0)),
        pl.BlockSpec((_RB, HF), lambda b, hb: (b, 0)),
        pl.BlockSpec((_RB, 128), lambda b, hb: (b, 0)),
        pl.BlockSpec((1, _RB, HF), lambda b, hb: (hb, b, 0)),
        pl.BlockSpec((1, _RB, HF), lambda b, hb: (hb, b, 0)),
    ],
    out_shape=[
        jax.ShapeDtypeStruct((NPAD, HF), jnp.float32),
        jax.ShapeDtypeStruct((NPAD, HF), jnp.float32),
        jax.ShapeDtypeStruct((NPAD, 128), jnp.float32),
        jax.ShapeDtypeStruct((NC, NPAD, HF), jnp.float32),
        jax.ShapeDtypeStruct((NC, NPAD, HF), jnp.float32),
    ],
)


def _make_combine(cst):
    def body(yp, z, h, dvb, dv2b, ho, zo):
        yt = yp[...][0] + z[...][0]
        ho[0] = h[...][0] + cst * (dvb[...] * yt)
        zo[0] = dv2b[...] * yt

    st = lambda: pl.BlockSpec((1, _RB, HF), lambda b, hb: (hb, b, 0))
    fl = lambda: pl.BlockSpec((_RB, HF), lambda b, hb: (b, 0))
    return pl.pallas_call(
        body,
        grid=(_GRID, NC),
        in_specs=[st(), st(), st(), fl(), fl()],
        out_specs=[st(), st()],
        out_shape=[jax.ShapeDtypeStruct((NC, NPAD, HF), jnp.float32)] * 2,
    )


def _make_layer_end(alpha_next, has_res):
    def body(*refs):
        if has_res:
            h, W, bvec, res, dvb, xo, ho, zo = refs
        else:
            h, W, bvec, dvb, xo, ho, zo = refs
        v = jnp.dot(h[...], W[...], preferred_element_type=jnp.float32)
        v = jnp.maximum(v + bvec[...], 0.0)
        if has_res:
            v = v + res[...]
        xo[...] = v
        ho[...] = alpha_next * v
        zo[...] = dvb[...] * v

    full = lambda: pl.BlockSpec((_RB, 128), lambda b: (b, 0))
    in_specs = [
        full(),
        pl.BlockSpec((128, 128), lambda b: (0, 0)),
        pl.BlockSpec((1, 128), lambda b: (0, 0)),
    ]
    if has_res:
        in_specs.append(full())
    in_specs.append(full())
    return pl.pallas_call(
        body,
        grid=(_GRID,),
        in_specs=in_specs,
        out_specs=[full() for _ in range(3)],
        out_shape=[jax.ShapeDtypeStruct((NPAD, 128), jnp.float32)] * 3,
    )


def _head_body(h, Wm, bm, pcol, prow, o):
    hm = jnp.dot(h[...], Wm[...], preferred_element_type=jnp.float32) + bm[...]
    sraw = jnp.dot(hm, pcol[...], preferred_element_type=jnp.float32)
    pv = prow[...]
    s = jnp.tanh(sraw / jnp.sqrt(jnp.sum(pv * pv)))
    ridx = lax.broadcasted_iota(jnp.int32, (NPAD, 1), 0)
    s = jnp.where(ridx < N, s, -jnp.inf)
    acc0 = jnp.full((1, 128), -jnp.inf, jnp.float32)

    def it(_, carry):
        sc, acc = carry
        m = jnp.max(sc)
        mask = sc == m
        contrib = jnp.max(jnp.where(mask, m * hm, -jnp.inf), axis=0,
                          keepdims=True)
        acc = jnp.maximum(acc, contrib)
        sc = jnp.where(mask, -jnp.inf, sc)
        return sc, acc

    _, acc = lax.fori_loop(0, 30, it, (s, acc0))
    o[...] = acc


_head = pl.pallas_call(
    _head_body,
    out_shape=jax.ShapeDtypeStruct((1, 128), jnp.float32),
)


def kernel(x, edge_index, W0, b0, W1, b1, W2, b2, W3, b3, W4, b4, Wm, bm, p):
    row = edge_index[0].astype(jnp.int32)
    col = edge_index[1].astype(jnp.int32)
    fill = jnp.full((EPAD - E,), DUMMY, jnp.int32)
    rowt = jnp.concatenate([row, fill]).reshape(NS, NPH, CPH, CH)
    colt = jnp.concatenate([col, fill]).reshape(NS, NPH, CPH, CH)

    ones3 = jnp.ones((NC, NPAD, HF), jnp.float32)
    degp = _sc_hop(ones3, rowt, colt)
    x4A = jnp.pad(x[:, :4], ((0, NPAD - N), (0, HF - 4)))
    x4st = jnp.stack([x4A, jnp.zeros_like(x4A)])
    dvb, dv2b, dvbf, hst, zst = _prep(degp[0], x4st)

    W0p = jnp.pad(W0, ((0, 124), (0, 0)))
    Ws = [W0p, W1, W2, W3, W4]
    bvs = [b0, b1, b2, b3, b4]

    def run_layer(hst, zst, comb):
        def step(_, hz):
            h, z = hz
            yp = _sc_hop(z, rowt, colt)
            h, z = comb(yp, z, h, dvb, dv2b)
            return h, z

        return lax.fori_loop(0, K, step, (hst, zst))

    def split(a):
        return jnp.stack([a[:, :HF], a[:, HF:]])

    xs = [None]
    for li in range(5):
        hst, zst = run_layer(hst, zst, _make_combine((1 - ALPHA[li]) / K))
        hfull = jnp.concatenate([hst[0], hst[1]], axis=1)
        alpha_next = ALPHA[li + 1] if li < 4 else 0.0
        has_res = li >= 2
        args = [hfull, Ws[li], bvs[li].reshape(1, 128)]
        if has_res:
            args += [xs[li]]
        args += [dvbf]
        xn, hn, zn = _make_layer_end(alpha_next, has_res)(*args)
        hst = split(hn)
        zst = split(zn)
        xs.append(xn)

    return _head(xs[5], Wm, bm.reshape(1, 128), p.reshape(128, 1),
                 p.reshape(1, 128))


# speedup vs baseline: 1.4871x; 1.4871x over previous
import jax
import jax.numpy as jnp
from jax import lax
from jax.experimental import pallas as pl
from jax.experimental.pallas import tpu as pltpu
from jax.experimental.pallas import tpu_sc as plsc

N = 10000
E = 320000
K = 20
ALPHA = [0.7, 0.7, 0.7 / 2.0, 0.7 / 3.0, 0.7 / 4.0]

NPAD = 10240
NC, NS = 2, 16
CH = 64
HF = 64
NPH = 2
CPH = 158
EPT = NPH * CPH * CH
EPAD = NS * EPT
RPS = NPAD // NS
DUMMY = NPAD - 1
SLOTS = 6
LAG = 3

_mesh = plsc.VectorSubcoreMesh(core_axis_name="c", subcore_axis_name="s")


def _sc_hop_body(z3, rowt, colt, ypart, row_v, col_v,
                 b0, b1, b2, b3, b4, b5,
                 g0, g1, g2, g3, g4, g5,
                 s0, s1, s2, s3, s4, s5,
                 y_sh, z_sh):
    c = lax.axis_index("c")
    s = lax.axis_index("s")
    bufs = [b0, b1, b2, b3, b4, b5]
    gsem = [g0, g1, g2, g3, g4, g5]
    ssem = [s0, s1, s2, s3, s4, s5]
    ztab = z3.at[c]

    def drain_gather(t):
        pltpu.make_async_copy(z_sh.at[pl.ds(0, CH)], bufs[t], gsem[t]).wait()

    def drain_scatter(t):
        pltpu.make_async_copy(bufs[t], y_sh.at[pl.ds(0, CH)], ssem[t]).wait()

    zero16 = jnp.zeros((16,), jnp.float32)

    @pl.loop(0, CH)
    def _zero(i):
        for g in range(HF // 16):
            b0[i, pl.ds(g * 16, 16)] = zero16

    zdesc = [pltpu.async_copy(b0, y_sh.at[pl.ds(s * RPS + q * CH, CH)],
                              ssem[q % SLOTS])
             for q in range(RPS // CH)]
    pltpu.sync_copy(ztab.at[pl.ds(s * RPS, RPS)], z_sh.at[pl.ds(s * RPS, RPS)])
    for dsc in zdesc:
        dsc.wait()
    plsc.subcore_barrier()

    for ph in range(NPH):
        pltpu.sync_copy(rowt.at[s].at[ph], row_v)
        pltpu.sync_copy(colt.at[s].at[ph], col_v)

        @pl.loop(0, (CPH + LAG + SLOTS - 1) // SLOTS + 1)
        def _grp(g):
            for t in range(SLOTS):
                j = g * SLOTS + t

                @pl.when(j <= CPH - 1)
                def _gather():
                    @pl.when(j >= SLOTS)
                    def _w():
                        drain_scatter(t)

                    pltpu.async_copy(z_sh.at[row_v.at[j]], bufs[t], gsem[t])

                i = j - LAG
                u = (t - LAG) % SLOTS

                @pl.when(jnp.logical_and(i >= 0, i <= CPH - 1))
                def _scatter():
                    drain_gather(u)
                    pltpu.async_copy(bufs[u], y_sh.at[col_v.at[i]],
                                     ssem[u], add=True)

        for t in range(SLOTS):
            drain_scatter(t)

    plsc.subcore_barrier()
    for q in range(RPS // CH):
        off = s * RPS + q * CH
        pltpu.sync_copy(y_sh.at[pl.ds(off, CH)], b1)
        pltpu.sync_copy(b1, ypart.at[c].at[pl.ds(off, CH)])


_sc_hop = pl.kernel(
    _sc_hop_body,
    out_type=jax.ShapeDtypeStruct((NC, NPAD, HF), jnp.float32),
    mesh=_mesh,
    compiler_params=pltpu.CompilerParams(use_tc_tiling_on_sc=False),
    scratch_types=(
        [
            pltpu.VMEM((CPH, CH), jnp.int32),
            pltpu.VMEM((CPH, CH), jnp.int32),
        ]
        + [pltpu.VMEM((CH, HF), jnp.float32)] * SLOTS
        + [pltpu.SemaphoreType.DMA] * (2 * SLOTS)
        + [pltpu.VMEM_SHARED((NPAD, HF), jnp.float32)] * 2
    ),
)


_RB = 1280
_GRID = NPAD // _RB


def _prep_body(dp, x4st, dvb, dv2b, dvbf, h0, z0):
    b = pl.program_id(0)
    deg = dp[...][:, 0:1] + 1.0
    dv = lax.rsqrt(deg)
    ridx = lax.broadcasted_iota(jnp.int32, (_RB, 1), 0) + b * _RB
    dv = jnp.where(ridx < N, dv, 0.0)
    dvb[...] = jnp.broadcast_to(dv, (_RB, HF))
    dv2b[...] = jnp.broadcast_to(dv * dv, (_RB, HF))
    dvbf[...] = jnp.broadcast_to(dv, (_RB, 128))
    xb = x4st[...][0]
    h0[0] = ALPHA[0] * xb
    z0[0] = dv * xb


_prep = pl.pallas_call(
    _prep_body,
    grid=(_GRID, NC),
    in_specs=[
        pl.BlockSpec((_RB, HF), lambda b, hb: (b, 0)),
        pl.BlockSpec((1, _RB, HF), lambda b, hb: (hb, b, 0)),
    ],
    out_specs=[
        pl.BlockSpec((_RB, HF), lambda b, hb: (b, 0)),
        pl.BlockSpec((_RB, HF), lambda b, hb: (b, 0)),
        pl.BlockSpec((_RB, 128), lambda b, hb: (b, 0)),
        pl.BlockSpec((1, _RB, HF), lambda b, hb: (hb, b, 0)),
        pl.BlockSpec((1, _RB, HF), lambda b, hb: (hb, b, 0)),
    ],
    out_shape=[
        jax.ShapeDtypeStruct((NPAD, HF), jnp.float32),
        jax.ShapeDtypeStruct((NPAD, HF), jnp.float32),
        jax.ShapeDtypeStruct((NPAD, 128), jnp.float32),
        jax.ShapeDtypeStruct((NC, NPAD, HF), jnp.float32),
        jax.ShapeDtypeStruct((NC, NPAD, HF), jnp.float32),
    ],
)


def _make_combine(cst):
    def body(yp, z, h, dvb, dv2b, ho, zo):
        yt = yp[...][0] + z[...][0]
        ho[0] = h[...][0] + cst * (dvb[...] * yt)
        zo[0] = dv2b[...] * yt

    st = lambda: pl.BlockSpec((1, _RB, HF), lambda b, hb: (hb, b, 0))
    fl = lambda: pl.BlockSpec((_RB, HF), lambda b, hb: (b, 0))
    return pl.pallas_call(
        body,
        grid=(_GRID, NC),
        in_specs=[st(), st(), st(), fl(), fl()],
        out_specs=[st(), st()],
        out_shape=[jax.ShapeDtypeStruct((NC, NPAD, HF), jnp.float32)] * 2,
    )


def _make_layer_end(alpha_next, has_res):
    def body(*refs):
        if has_res:
            h, W, bvec, res, dvb, xo, ho, zo = refs
        else:
            h, W, bvec, dvb, xo, ho, zo = refs
        v = jnp.dot(h[...], W[...], preferred_element_type=jnp.float32)
        v = jnp.maximum(v + bvec[...], 0.0)
        if has_res:
            v = v + res[...]
        xo[...] = v
        ho[...] = alpha_next * v
        zo[...] = dvb[...] * v

    full = lambda: pl.BlockSpec((_RB, 128), lambda b: (b, 0))
    in_specs = [
        full(),
        pl.BlockSpec((128, 128), lambda b: (0, 0)),
        pl.BlockSpec((1, 128), lambda b: (0, 0)),
    ]
    if has_res:
        in_specs.append(full())
    in_specs.append(full())
    return pl.pallas_call(
        body,
        grid=(_GRID,),
        in_specs=in_specs,
        out_specs=[full() for _ in range(3)],
        out_shape=[jax.ShapeDtypeStruct((NPAD, 128), jnp.float32)] * 3,
    )


def _head_body(h, Wm, bm, pcol, prow, o):
    hm = jnp.dot(h[...], Wm[...], preferred_element_type=jnp.float32) + bm[...]
    sraw = jnp.dot(hm, pcol[...], preferred_element_type=jnp.float32)
    pv = prow[...]
    s = jnp.tanh(sraw / jnp.sqrt(jnp.sum(pv * pv)))
    ridx = lax.broadcasted_iota(jnp.int32, (NPAD, 1), 0)
    s = jnp.where(ridx < N, s, -jnp.inf)
    acc0 = jnp.full((1, 128), -jnp.inf, jnp.float32)

    def it(_, carry):
        sc, acc = carry
        m = jnp.max(sc)
        mask = sc == m
        contrib = jnp.max(jnp.where(mask, m * hm, -jnp.inf), axis=0,
                          keepdims=True)
        acc = jnp.maximum(acc, contrib)
        sc = jnp.where(mask, -jnp.inf, sc)
        return sc, acc

    _, acc = lax.fori_loop(0, 30, it, (s, acc0))
    o[...] = acc


_head = pl.pallas_call(
    _head_body,
    out_shape=jax.ShapeDtypeStruct((1, 128), jnp.float32),
)


def kernel(x, edge_index, W0, b0, W1, b1, W2, b2, W3, b3, W4, b4, Wm, bm, p):
    row = edge_index[0].astype(jnp.int32)
    col = edge_index[1].astype(jnp.int32)
    fill = jnp.full((EPAD - E,), DUMMY, jnp.int32)
    rowt = jnp.concatenate([row, fill]).reshape(NS, NPH, CPH, CH)
    colt = jnp.concatenate([col, fill]).reshape(NS, NPH, CPH, CH)

    ones3 = jnp.ones((NC, NPAD, HF), jnp.float32)
    degp = _sc_hop(ones3, rowt, colt)
    x4A = jnp.pad(x[:, :4], ((0, NPAD - N), (0, HF - 4)))
    x4st = jnp.stack([x4A, jnp.zeros_like(x4A)])
    dvb, dv2b, dvbf, hst, zst = _prep(degp[0], x4st)

    W0p = jnp.pad(W0, ((0, 124), (0, 0)))
    Ws = [W0p, W1, W2, W3, W4]
    bvs = [b0, b1, b2, b3, b4]

    def run_layer(hst, zst, comb):
        def step(_, hz):
            h, z = hz
            yp = _sc_hop(z, rowt, colt)
            h, z = comb(yp, z, h, dvb, dv2b)
            return h, z

        return lax.fori_loop(0, K, step, (hst, zst))

    def split(a):
        return jnp.stack([a[:, :HF], a[:, HF:]])

    xs = [None]
    for li in range(5):
        hst, zst = run_layer(hst, zst, _make_combine((1 - ALPHA[li]) / K))
        hfull = jnp.concatenate([hst[0], hst[1]], axis=1)
        alpha_next = ALPHA[li + 1] if li < 4 else 0.0
        has_res = li >= 2
        args = [hfull, Ws[li], bvs[li].reshape(1, 128)]
        if has_res:
            args += [xs[li]]
        args += [dvbf]
        xn, hn, zn = _make_layer_end(alpha_next, has_res)(*args)
        hst = split(hn)
        zst = split(zn)
        xs.append(xn)

    return _head(xs[5], Wm, bm.reshape(1, 128), p.reshape(128, 1),
                 p.reshape(1, 128))


# speedup vs baseline: 1.5014x; 1.0096x over previous
import jax
import jax.numpy as jnp
from jax import lax
from jax.experimental import pallas as pl
from jax.experimental.pallas import tpu as pltpu
from jax.experimental.pallas import tpu_sc as plsc

N = 10000
E = 320000
K = 20
ALPHA = [0.7, 0.7, 0.7 / 2.0, 0.7 / 3.0, 0.7 / 4.0]

NPAD = 10240
NC, NS = 2, 16
CH = 64
HF = 64
NPH = 2
CPH = 158
EPT = NPH * CPH * CH
EPAD = NS * EPT
RPS = NPAD // NS
DUMMY = NPAD - 1
SLOTS = 6
LAG = 3

_mesh = plsc.VectorSubcoreMesh(core_axis_name="c", subcore_axis_name="s")


def _sc_hop_body(z3, rowt, colt, ypart, row_v, col_v,
                 b0, b1, b2, b3, b4, b5,
                 g0, g1, g2, g3, g4, g5,
                 s0, s1, s2, s3, s4, s5,
                 y_sh, z_sh):
    c = lax.axis_index("c")
    s = lax.axis_index("s")
    bufs = [b0, b1, b2, b3, b4, b5]
    gsem = [g0, g1, g2, g3, g4, g5]
    ssem = [s0, s1, s2, s3, s4, s5]
    ztab = z3.at[c]

    def drain_gather(t):
        pltpu.make_async_copy(z_sh.at[pl.ds(0, CH)], bufs[t], gsem[t]).wait()

    def drain_scatter(t):
        pltpu.make_async_copy(bufs[t], y_sh.at[pl.ds(0, CH)], ssem[t]).wait()

    zero16 = jnp.zeros((16,), jnp.float32)

    @pl.loop(0, CH)
    def _zero(i):
        for g in range(HF // 16):
            b0[i, pl.ds(g * 16, 16)] = zero16

    zdesc = [pltpu.async_copy(b0, y_sh.at[pl.ds(s * RPS + q * CH, CH)],
                              ssem[q % SLOTS])
             for q in range(RPS // CH)]
    pltpu.sync_copy(ztab.at[pl.ds(s * RPS, RPS)], z_sh.at[pl.ds(s * RPS, RPS)])
    for dsc in zdesc:
        dsc.wait()
    plsc.subcore_barrier()

    for ph in range(NPH):
        pltpu.sync_copy(rowt.at[s].at[ph], row_v)
        pltpu.sync_copy(colt.at[s].at[ph], col_v)

        @pl.loop(0, (CPH + LAG + SLOTS - 1) // SLOTS + 1)
        def _grp(g):
            for t in range(SLOTS):
                j = g * SLOTS + t

                @pl.when(j <= CPH - 1)
                def _gather():
                    @pl.when(j >= SLOTS)
                    def _w():
                        drain_scatter(t)

                    pltpu.async_copy(z_sh.at[row_v.at[j]], bufs[t], gsem[t])

                i = j - LAG
                u = (t - LAG) % SLOTS

                @pl.when(jnp.logical_and(i >= 0, i <= CPH - 1))
                def _scatter():
                    drain_gather(u)
                    pltpu.async_copy(bufs[u], y_sh.at[col_v.at[i]],
                                     ssem[u], add=True)

        for t in range(SLOTS):
            drain_scatter(t)

    plsc.subcore_barrier()
    nq = RPS // CH
    outd = {}
    for q in range(nq):
        t = q % SLOTS
        if q >= SLOTS:
            outd[q - SLOTS].wait()
        off = s * RPS + q * CH
        pltpu.async_copy(y_sh.at[pl.ds(off, CH)], bufs[t], gsem[t]).wait()
        outd[q] = pltpu.async_copy(bufs[t], ypart.at[c].at[pl.ds(off, CH)],
                                   ssem[t])
    for q in range(max(0, nq - SLOTS), nq):
        outd[q].wait()


_sc_hop = pl.kernel(
    _sc_hop_body,
    out_type=jax.ShapeDtypeStruct((NC, NPAD, HF), jnp.float32),
    mesh=_mesh,
    compiler_params=pltpu.CompilerParams(use_tc_tiling_on_sc=False),
    scratch_types=(
        [
            pltpu.VMEM((CPH, CH), jnp.int32),
            pltpu.VMEM((CPH, CH), jnp.int32),
        ]
        + [pltpu.VMEM((CH, HF), jnp.float32)] * SLOTS
        + [pltpu.SemaphoreType.DMA] * (2 * SLOTS)
        + [pltpu.VMEM_SHARED((NPAD, HF), jnp.float32)] * 2
    ),
)


_RB = 1280
_GRID = NPAD // _RB


def _prep_body(dp, x4st, dvb, dv2b, dvbf, h0, z0):
    b = pl.program_id(0)
    deg = dp[...][:, 0:1] + 1.0
    dv = lax.rsqrt(deg)
    ridx = lax.broadcasted_iota(jnp.int32, (_RB, 1), 0) + b * _RB
    dv = jnp.where(ridx < N, dv, 0.0)
    dvb[...] = jnp.broadcast_to(dv, (_RB, HF))
    dv2b[...] = jnp.broadcast_to(dv * dv, (_RB, HF))
    dvbf[...] = jnp.broadcast_to(dv, (_RB, 128))
    xb = x4st[...][0]
    h0[0] = ALPHA[0] * xb
    z0[0] = dv * xb


_prep = pl.pallas_call(
    _prep_body,
    grid=(_GRID, NC),
    in_specs=[
        pl.BlockSpec((_RB, HF), lambda b, hb: (b, 0)),
        pl.BlockSpec((1, _RB, HF), lambda b, hb: (hb, b, 0)),
    ],
    out_specs=[
        pl.BlockSpec((_RB, HF), lambda b, hb: (b, 0)),
        pl.BlockSpec((_RB, HF), lambda b, hb: (b, 0)),
        pl.BlockSpec((_RB, 128), lambda b, hb: (b, 0)),
        pl.BlockSpec((1, _RB, HF), lambda b, hb: (hb, b, 0)),
        pl.BlockSpec((1, _RB, HF), lambda b, hb: (hb, b, 0)),
    ],
    out_shape=[
        jax.ShapeDtypeStruct((NPAD, HF), jnp.float32),
        jax.ShapeDtypeStruct((NPAD, HF), jnp.float32),
        jax.ShapeDtypeStruct((NPAD, 128), jnp.float32),
        jax.ShapeDtypeStruct((NC, NPAD, HF), jnp.float32),
        jax.ShapeDtypeStruct((NC, NPAD, HF), jnp.float32),
    ],
)


def _make_combine(cst):
    def body(yp, z, h, dvb, dv2b, ho, zo):
        yt = yp[...][0] + z[...][0]
        ho[0] = h[...][0] + cst * (dvb[...] * yt)
        zo[0] = dv2b[...] * yt

    st = lambda: pl.BlockSpec((1, _RB, HF), lambda b, hb: (hb, b, 0))
    fl = lambda: pl.BlockSpec((_RB, HF), lambda b, hb: (b, 0))
    return pl.pallas_call(
        body,
        grid=(_GRID, NC),
        in_specs=[st(), st(), st(), fl(), fl()],
        out_specs=[st(), st()],
        out_shape=[jax.ShapeDtypeStruct((NC, NPAD, HF), jnp.float32)] * 2,
    )


def _make_layer_end(alpha_next, has_res):
    def body(*refs):
        if has_res:
            h, W, bvec, res, dvb, xo, ho, zo = refs
        else:
            h, W, bvec, dvb, xo, ho, zo = refs
        v = jnp.dot(h[...], W[...], preferred_element_type=jnp.float32)
        v = jnp.maximum(v + bvec[...], 0.0)
        if has_res:
            v = v + res[...]
        xo[...] = v
        ho[...] = alpha_next * v
        zo[...] = dvb[...] * v

    full = lambda: pl.BlockSpec((_RB, 128), lambda b: (b, 0))
    in_specs = [
        full(),
        pl.BlockSpec((128, 128), lambda b: (0, 0)),
        pl.BlockSpec((1, 128), lambda b: (0, 0)),
    ]
    if has_res:
        in_specs.append(full())
    in_specs.append(full())
    return pl.pallas_call(
        body,
        grid=(_GRID,),
        in_specs=in_specs,
        out_specs=[full() for _ in range(3)],
        out_shape=[jax.ShapeDtypeStruct((NPAD, 128), jnp.float32)] * 3,
    )


def _head_body(h, Wm, bm, pcol, prow, o):
    hm = jnp.dot(h[...], Wm[...], preferred_element_type=jnp.float32) + bm[...]
    sraw = jnp.dot(hm, pcol[...], preferred_element_type=jnp.float32)
    pv = prow[...]
    s = jnp.tanh(sraw / jnp.sqrt(jnp.sum(pv * pv)))
    ridx = lax.broadcasted_iota(jnp.int32, (NPAD, 1), 0)
    s = jnp.where(ridx < N, s, -jnp.inf)
    acc0 = jnp.full((1, 128), -jnp.inf, jnp.float32)

    def it(_, carry):
        sc, acc = carry
        m = jnp.max(sc)
        mask = sc == m
        contrib = jnp.max(jnp.where(mask, m * hm, -jnp.inf), axis=0,
                          keepdims=True)
        acc = jnp.maximum(acc, contrib)
        sc = jnp.where(mask, -jnp.inf, sc)
        return sc, acc

    _, acc = lax.fori_loop(0, 30, it, (s, acc0))
    o[...] = acc


_head = pl.pallas_call(
    _head_body,
    out_shape=jax.ShapeDtypeStruct((1, 128), jnp.float32),
)


def kernel(x, edge_index, W0, b0, W1, b1, W2, b2, W3, b3, W4, b4, Wm, bm, p):
    row = edge_index[0].astype(jnp.int32)
    col = edge_index[1].astype(jnp.int32)
    fill = jnp.full((EPAD - E,), DUMMY, jnp.int32)
    rowt = jnp.concatenate([row, fill]).reshape(NS, NPH, CPH, CH)
    colt = jnp.concatenate([col, fill]).reshape(NS, NPH, CPH, CH)

    ones3 = jnp.ones((NC, NPAD, HF), jnp.float32)
    degp = _sc_hop(ones3, rowt, colt)
    x4A = jnp.pad(x[:, :4], ((0, NPAD - N), (0, HF - 4)))
    x4st = jnp.stack([x4A, jnp.zeros_like(x4A)])
    dvb, dv2b, dvbf, hst, zst = _prep(degp[0], x4st)

    W0p = jnp.pad(W0, ((0, 124), (0, 0)))
    Ws = [W0p, W1, W2, W3, W4]
    bvs = [b0, b1, b2, b3, b4]

    def run_layer(hst, zst, comb):
        def step(_, hz):
            h, z = hz
            yp = _sc_hop(z, rowt, colt)
            h, z = comb(yp, z, h, dvb, dv2b)
            return h, z

        return lax.fori_loop(0, K, step, (hst, zst))

    def split(a):
        return jnp.stack([a[:, :HF], a[:, HF:]])

    xs = [None]
    for li in range(5):
        hst, zst = run_layer(hst, zst, _make_combine((1 - ALPHA[li]) / K))
        hfull = jnp.concatenate([hst[0], hst[1]], axis=1)
        alpha_next = ALPHA[li + 1] if li < 4 else 0.0
        has_res = li >= 2
        args = [hfull, Ws[li], bvs[li].reshape(1, 128)]
        if has_res:
            args += [xs[li]]
        args += [dvbf]
        xn, hn, zn = _make_layer_end(alpha_next, has_res)(*args)
        hst = split(hn)
        zst = split(zn)
        xs.append(xn)

    return _head(xs[5], Wm, bm.reshape(1, 128), p.reshape(128, 1),
                 p.reshape(1, 128))
